# speedup vs baseline: 1.4865x; 1.4865x over previous
import functools

import jax
import jax.numpy as jnp
from jax import lax
from jax.experimental import pallas as pl
from jax.experimental.pallas import tpu as pltpu
from jax.experimental.pallas import tpu_sc as plsc

_HIDDEN = 1024
_NUM_CORES = 2
_NUM_SUBCORES = 16
_NW = _NUM_CORES * _NUM_SUBCORES


def _embed_body(b_per_w, chunk, tokens_hbm, table_hbm, out_hbm,
                idx_v, row_buf, gsem):
    wid = lax.axis_index("s") * _NUM_CORES + lax.axis_index("c")
    base = wid * b_per_w
    pltpu.sync_copy(tokens_hbm.at[pl.ds(base, b_per_w)], idx_v)
    nchunk = b_per_w // chunk

    def step(c, carry):
        off = c * chunk
        pltpu.async_copy(
            table_hbm.at[idx_v.at[pl.ds(off, chunk)]], row_buf, gsem
        ).wait()
        pltpu.sync_copy(row_buf, out_hbm.at[pl.ds(base + off, chunk)])
        return carry

    lax.fori_loop(0, nchunk, step, 0, unroll=False)


def kernel(tokens, embedding):
    b = tokens.size
    b_per_w = b // _NW
    chunk = 32
    flat = tokens.reshape(b)
    mesh = plsc.VectorSubcoreMesh(core_axis_name="c", subcore_axis_name="s")
    out = pl.kernel(
        functools.partial(_embed_body, b_per_w, chunk),
        out_type=jax.ShapeDtypeStruct((b, _HIDDEN), jnp.float32),
        mesh=mesh,
        scratch_types=[
            pltpu.VMEM((b_per_w,), jnp.int32),
            pltpu.VMEM((chunk, _HIDDEN), jnp.float32),
            pltpu.SemaphoreType.DMA,
        ],
    )(flat, embedding)
    return out.reshape(tokens.shape + (_HIDDEN,))


# speedup vs baseline: 1.6818x; 1.1314x over previous
import functools

import jax
import jax.numpy as jnp
from jax import lax
from jax.experimental import pallas as pl
from jax.experimental.pallas import tpu as pltpu
from jax.experimental.pallas import tpu_sc as plsc

_HIDDEN = 1024
_NUM_CORES = 2
_NUM_SUBCORES = 16
_NW = _NUM_CORES * _NUM_SUBCORES


def _embed_body(b_per_w, chunk, tokens_hbm, table_hbm, out_hbm,
                idx_v, buf0, buf1, gsem0, gsem1, ssem0, ssem1):
    wid = lax.axis_index("s") * _NUM_CORES + lax.axis_index("c")
    base = wid * b_per_w
    pltpu.sync_copy(tokens_hbm.at[pl.ds(base, b_per_w)], idx_v)
    nchunk = b_per_w // chunk
    npair = nchunk // 2

    def start_gather(c, buf, sem):
        pltpu.async_copy(
            table_hbm.at[idx_v.at[pl.ds(c * chunk, chunk)]], buf, sem)

    def wait_gather(c, buf, sem):
        pltpu.make_async_copy(
            table_hbm.at[idx_v.at[pl.ds(c * chunk, chunk)]], buf, sem).wait()

    def start_store(c, buf, sem):
        pltpu.async_copy(
            buf, out_hbm.at[pl.ds(base + c * chunk, chunk)], sem)

    def wait_store(c, buf, sem):
        pltpu.make_async_copy(
            buf, out_hbm.at[pl.ds(base + c * chunk, chunk)], sem).wait()

    start_gather(0, buf0, gsem0)
    start_gather(1, buf1, gsem1)

    def pair_step(p, carry):
        c0 = 2 * p
        wait_gather(c0, buf0, gsem0)
        start_store(c0, buf0, ssem0)
        wait_gather(c0 + 1, buf1, gsem1)
        start_store(c0 + 1, buf1, ssem1)
        wait_store(c0, buf0, ssem0)
        start_gather(c0 + 2, buf0, gsem0)
        wait_store(c0 + 1, buf1, ssem1)
        start_gather(c0 + 3, buf1, gsem1)
        return carry

    lax.fori_loop(0, npair - 1, pair_step, 0, unroll=False)

    c0 = 2 * (npair - 1)
    wait_gather(c0, buf0, gsem0)
    start_store(c0, buf0, ssem0)
    wait_gather(c0 + 1, buf1, gsem1)
    start_store(c0 + 1, buf1, ssem1)
    wait_store(c0, buf0, ssem0)
    wait_store(c0 + 1, buf1, ssem1)


def kernel(tokens, embedding):
    b = tokens.size
    b_per_w = b // _NW
    chunk = 32
    flat = tokens.reshape(b)
    mesh = plsc.VectorSubcoreMesh(core_axis_name="c", subcore_axis_name="s")
    out = pl.kernel(
        functools.partial(_embed_body, b_per_w, chunk),
        out_type=jax.ShapeDtypeStruct((b, _HIDDEN), jnp.float32),
        mesh=mesh,
        scratch_types=[
            pltpu.VMEM((b_per_w,), jnp.int32),
            pltpu.VMEM((chunk, _HIDDEN), jnp.float32),
            pltpu.VMEM((chunk, _HIDDEN), jnp.float32),
            pltpu.SemaphoreType.DMA,
            pltpu.SemaphoreType.DMA,
            pltpu.SemaphoreType.DMA,
            pltpu.SemaphoreType.DMA,
        ],
    )(flat, embedding)
    return out.reshape(tokens.shape + (_HIDDEN,))
